# baseline (speedup 1.0000x reference)
import jax
import jax.numpy as jnp
from jax import lax
from jax.experimental import pallas as pl
from jax.experimental.pallas import tpu as pltpu

N = 10000
E = 320000
D = 128
R = 20
NB = 10
Q = 1024

NP = 10240
CHUNK = 512


def _tc_body(s_ref, deg_ref, x_ref, batch_ref, comp_ref, basis_ref,
             root_ref, bias_ref, tgt_ref, out_ref, pooled):
  i = pl.program_id(0)

  @pl.when(i == 0)
  def _():
    pooled[...] = jnp.zeros_like(pooled)

  s = s_ref[...]
  deg = deg_ref[:, :, 0]
  inv = 1.0 / jnp.maximum(deg, 1.0)
  sm = s * inv[:, :, None]
  sm2 = sm.reshape(R, CHUNK * D)
  t = jax.lax.dot_general(comp_ref[...], sm2, (((0,), (0,)), ((), ())),
                          preferred_element_type=jnp.float32)
  agg = jnp.zeros((CHUNK, D), jnp.float32)
  for b in range(NB):
    agg = agg + jax.lax.dot_general(
        t[b].reshape(CHUNK, D), basis_ref[b], (((1,), (0,)), ((), ())),
        preferred_element_type=jnp.float32)
  out = agg + jax.lax.dot_general(
      x_ref[...], root_ref[...], (((1,), (0,)), ((), ())),
      preferred_element_type=jnp.float32) + bias_ref[...]
  out = jnp.maximum(out, 0.0)

  bids = batch_ref[0, 0, :].reshape(CHUNK, 1)
  qiota = jax.lax.broadcasted_iota(jnp.int32, (CHUNK, Q), 1)
  onehot = (bids == qiota).astype(jnp.float32)
  pooled[...] += jax.lax.dot_general(onehot, out, (((0,), (0,)), ((), ())),
                                     preferred_element_type=jnp.float32)

  @pl.when(i == (NP // CHUNK) - 1)
  def _():
    p = pooled[...]
    tgt = tgt_ref[...]
    num = jnp.sum(p * tgt, axis=1)
    pn = jnp.maximum(jnp.sqrt(jnp.sum(p * p, axis=1)), 1e-8)
    tn = jnp.maximum(jnp.sqrt(jnp.sum(tgt * tgt, axis=1)), 1e-8)
    out_ref[...] = (num / (pn * tn)).reshape(1, Q)


@jax.jit
def kernel(x, edge_index, edge_type, batch, target_embeds, comp, basis,
           root, bias):
  src = edge_index[0]
  dst = edge_index[1]
  x_pad = jnp.zeros((NP, D), jnp.float32).at[:N].set(x)
  batch_pad = jnp.full((NP,), Q, jnp.int32).at[:N].set(batch)
  batch3 = batch_pad.reshape(NP // CHUNK, 1, CHUNK)

  comb = edge_type * NP + dst
  s = jax.ops.segment_sum(x_pad[src], comb,
                          num_segments=R * NP).reshape(R, NP, D)
  degf = jax.ops.segment_sum(jnp.ones((E,), jnp.float32), comb,
                             num_segments=R * NP)
  deg = jnp.broadcast_to(degf.reshape(R, NP, 1), (R, NP, 16))

  nblocks = NP // CHUNK
  scores = pl.pallas_call(
      _tc_body,
      grid=(nblocks,),
      in_specs=[
          pl.BlockSpec((R, CHUNK, D), lambda i: (0, i, 0)),
          pl.BlockSpec((R, CHUNK, 16), lambda i: (0, i, 0)),
          pl.BlockSpec((CHUNK, D), lambda i: (i, 0)),
          pl.BlockSpec((1, 1, CHUNK), lambda i: (i, 0, 0)),
          pl.BlockSpec((R, NB), lambda i: (0, 0)),
          pl.BlockSpec((NB, D, D), lambda i: (0, 0, 0)),
          pl.BlockSpec((D, D), lambda i: (0, 0)),
          pl.BlockSpec((1, D), lambda i: (0, 0)),
          pl.BlockSpec((Q, D), lambda i: (0, 0)),
      ],
      out_specs=pl.BlockSpec((1, Q), lambda i: (0, 0)),
      out_shape=jax.ShapeDtypeStruct((1, Q), jnp.float32),
      scratch_shapes=[pltpu.VMEM((Q, D), jnp.float32)],
  )(s, deg, x_pad, batch3, comp, basis, root, bias.reshape(1, D),
    target_embeds)
  return scores.reshape(Q)
